# Initial kernel scaffold; baseline (speedup 1.0000x reference)
import jax
import jax.numpy as jnp
from jax.experimental import pallas as pl


def kernel(x, edge_index, eps1, W11, b11, W12, b12, eps2, W21, b21, W22, b22, bn1_g, bn1_b, bn2_g, bn2_b):
    raise NotImplementedError("write your pallas kernel here")


# speedup vs baseline: 4.7353x; 4.7353x over previous
import functools

import jax
import jax.numpy as jnp
from jax import lax
from jax.experimental import pallas as pl
from jax.experimental.pallas import tpu as pltpu
from jax.experimental.pallas import tpu_sc as plsc

N = 10000
D = 128
E = 320000
NC, NS = 2, 16
NW = NC * NS
CHUNK = 128
CPW = 79
EPW = CPW * CHUNK
E_PAD = NW * EPW
ACC_ROWS = 10112
RPT = ACC_ROWS // NS


def _sc_agg_body(feat_h, src_h, dst_h, zeros_h, out_h,
                 acc, src_v, dst_v, rows_v, sem):
    c = lax.axis_index("c")
    s = lax.axis_index("s")
    wid = c * NS + s
    pltpu.sync_copy(zeros_h, acc.at[pl.ds(s * RPT, RPT)])
    pltpu.sync_copy(src_h.at[wid], src_v)
    pltpu.sync_copy(dst_h.at[wid], dst_v)
    plsc.subcore_barrier()

    def step(j, carry):
        pltpu.async_copy(feat_h.at[src_v.at[j]], rows_v, sem).wait()
        pltpu.sync_copy(rows_v, acc.at[dst_v.at[j]], add=True)
        return carry

    lax.fori_loop(0, CPW, step, 0)
    plsc.subcore_barrier()
    pltpu.sync_copy(acc.at[pl.ds(s * RPT, RPT)],
                    out_h.at[c, pl.ds(s * RPT, RPT)])


def _sc_agg(feat, src3, dst3, zeros):
    mesh = plsc.VectorSubcoreMesh(
        core_axis_name="c", subcore_axis_name="s")
    fn = pl.kernel(
        _sc_agg_body,
        out_type=jax.ShapeDtypeStruct((NC, ACC_ROWS, D), jnp.float32),
        mesh=mesh,
        scratch_types=[
            pltpu.VMEM_SHARED((ACC_ROWS, D), jnp.float32),
            pltpu.VMEM((CPW, CHUNK), jnp.int32),
            pltpu.VMEM((CPW, CHUNK), jnp.int32),
            pltpu.VMEM((CHUNK, D), jnp.float32),
            pltpu.SemaphoreType.DMA,
        ],
    )
    return fn(feat, src3, dst3, zeros)


def _tc_layer_body(eps_r, x_r, agg_r, w1_r, b1_r, w2_r, b2_r, g_r, b_r, o_r):
    scale = 1.0 + eps_r[0, 0]
    h = scale * x_r[...] + agg_r[0, :N, :] + agg_r[1, :N, :]
    h = jnp.dot(h, w1_r[...], preferred_element_type=jnp.float32,
                precision=lax.Precision.DEFAULT) + b1_r[...]
    h = jnp.maximum(h, 0.0)
    h = jnp.dot(h, w2_r[...], preferred_element_type=jnp.float32,
                precision=lax.Precision.DEFAULT) + b2_r[...]
    mu = jnp.mean(h, axis=0, keepdims=True)
    var = jnp.mean((h - mu) ** 2, axis=0, keepdims=True)
    o = (h - mu) * lax.rsqrt(var + 1e-5) * g_r[...] + b_r[...]
    o_r[...] = jnp.maximum(o, 0.0)


def _tc_layer(xin, agg, eps, w1, b1, w2, b2, g, b):
    vmem = pl.BlockSpec(memory_space=pltpu.VMEM)
    return pl.pallas_call(
        _tc_layer_body,
        out_shape=jax.ShapeDtypeStruct((N, D), jnp.float32),
        in_specs=[pl.BlockSpec(memory_space=pltpu.SMEM)] + [vmem] * 8,
        out_specs=vmem,
    )(eps.reshape(1, 1), xin, agg, w1, b1.reshape(1, D), w2,
      b2.reshape(1, D), g.reshape(1, D), b.reshape(1, D))


def kernel(x, edge_index, eps1, W11, b11, W12, b12,
           eps2, W21, b21, W22, b22, bn1_g, bn1_b, bn2_g, bn2_b):
    src = edge_index[0].astype(jnp.int32)
    dst = edge_index[1].astype(jnp.int32)
    pad = E_PAD - E
    src3 = jnp.concatenate(
        [src, jnp.zeros((pad,), jnp.int32)]).reshape(NW, CPW, CHUNK)
    dst3 = jnp.concatenate(
        [dst, jnp.full((pad,), N, jnp.int32)]).reshape(NW, CPW, CHUNK)
    zeros = jnp.zeros((RPT, D), jnp.float32)

    agg1 = _sc_agg(x, src3, dst3, zeros)
    h1 = _tc_layer(x, agg1, eps1, W11, b11, W12, b12, bn1_g, bn1_b)
    agg2 = _sc_agg(h1, src3, dst3, zeros)
    return _tc_layer(h1, agg2, eps2, W21, b21, W22, b22, bn2_g, bn2_b)


# speedup vs baseline: 5.4249x; 1.1456x over previous
import jax
import jax.numpy as jnp
from jax import lax
from jax.experimental import pallas as pl
from jax.experimental.pallas import tpu as pltpu
from jax.experimental.pallas import tpu_sc as plsc

N = 10000
D = 128
HD = D // 2
E = 320000
NC, NS = 2, 16
CHUNK = 128
CPW = 160
NB = 4
EPW = CPW * CHUNK
ACC_ROWS = 10112
RPT = ACC_ROWS // NS


def _sc_agg_body(feat_h, src_h, dst_h, zeros_h, out_h,
                 acc, src_v, dst_v, rows_all, sem0, sem1, sem2, sem3):
    rows = tuple(rows_all.at[pl.ds(k * CHUNK, CHUNK)] for k in range(NB))
    sems = (sem0, sem1, sem2, sem3)
    c = lax.axis_index("c")
    s = lax.axis_index("s")
    pltpu.sync_copy(zeros_h, rows[0])
    for q in range(RPT // CHUNK):
        pltpu.sync_copy(rows[0], acc.at[pl.ds(s * RPT + q * CHUNK, CHUNK)])
    rem = RPT % CHUNK
    if rem:
        pltpu.sync_copy(rows[0].at[pl.ds(0, rem)],
                        acc.at[pl.ds(s * RPT + RPT - rem, rem)])
    pltpu.sync_copy(src_h.at[c, s], src_v)
    pltpu.sync_copy(dst_h.at[s], dst_v)
    plsc.subcore_barrier()

    def visit(j, p, k, reissue):
        pltpu.make_async_copy(feat_h.at[src_v.at[j]],
                              rows[2 * p + k], sems[2 * p + k]).wait()
        pltpu.sync_copy(rows[2 * p + k], acc.at[dst_v.at[j]], add=True)
        if reissue:
            q = 1 - p
            pltpu.async_copy(feat_h.at[src_v.at[j + 2]],
                             rows[2 * q + k], sems[2 * q + k])

    for k in range(2):
        pltpu.async_copy(feat_h.at[src_v.at[k]], rows[k], sems[k])

    def iteration(i, carry):
        for k in range(2):
            visit(4 * i + k, 0, k, True)
        for k in range(2):
            visit(4 * i + 2 + k, 1, k, True)
        return carry

    lax.fori_loop(0, CPW // 4 - 1, iteration, 0)
    i_last = CPW // 4 - 1
    for k in range(2):
        visit(4 * i_last + k, 0, k, True)
    for k in range(2):
        visit(4 * i_last + 2 + k, 1, k, False)

    plsc.subcore_barrier()
    for off, sz in ([(0, NB * CHUNK), (NB * CHUNK, RPT - NB * CHUNK)]
                    if RPT > NB * CHUNK else [(0, RPT)]):
        pltpu.sync_copy(acc.at[pl.ds(s * RPT + off, sz)],
                        rows_all.at[pl.ds(0, sz)])
        pltpu.sync_copy(rows_all.at[pl.ds(0, sz)],
                        out_h.at[c, pl.ds(s * RPT + off, sz)])


def _sc_agg(feat2, src4, dst3, zeros):
    mesh = plsc.VectorSubcoreMesh(
        core_axis_name="c", subcore_axis_name="s")
    fn = pl.kernel(
        _sc_agg_body,
        out_type=jax.ShapeDtypeStruct((NC, ACC_ROWS, HD), jnp.float32),
        mesh=mesh,
        scratch_types=[
            pltpu.VMEM_SHARED((ACC_ROWS, HD), jnp.float32),
            pltpu.VMEM((CPW, CHUNK), jnp.int32),
            pltpu.VMEM((CPW, CHUNK), jnp.int32),
            pltpu.VMEM((NB * CHUNK, HD), jnp.float32),
        ] + [pltpu.SemaphoreType.DMA] * NB,
        compiler_params=pltpu.CompilerParams(use_tc_tiling_on_sc=False),
    )
    return fn(feat2, src4, dst3, zeros)


def _tc_layer_body(split_in, split_out):
    def body(eps_r, x_r, agg_r, w1_r, b1_r, w2_r, b2_r, g_r, b_r, o_r):
        scale = 1.0 + eps_r[0, 0]
        if split_in:
            xv = jnp.concatenate([x_r[0], x_r[1]], axis=1)
        else:
            xv = x_r[...]
        agg = jnp.concatenate([agg_r[0, :N, :], agg_r[1, :N, :]], axis=1)
        h = scale * xv + agg
        h = jnp.dot(h, w1_r[...], preferred_element_type=jnp.float32,
                    precision=lax.Precision.DEFAULT) + b1_r[...]
        h = jnp.maximum(h, 0.0)
        h = jnp.dot(h, w2_r[...], preferred_element_type=jnp.float32,
                    precision=lax.Precision.DEFAULT) + b2_r[...]
        mu = jnp.mean(h, axis=0, keepdims=True)
        var = jnp.mean((h - mu) ** 2, axis=0, keepdims=True)
        o = (h - mu) * lax.rsqrt(var + 1e-5) * g_r[...] + b_r[...]
        o = jnp.maximum(o, 0.0)
        if split_out:
            o_r[0] = o[:, :HD]
            o_r[1] = o[:, HD:]
        else:
            o_r[...] = o
    return body


def _tc_layer(xin, agg, eps, w1, b1, w2, b2, g, b, split_in, split_out):
    vmem = pl.BlockSpec(memory_space=pltpu.VMEM)
    out_shape = (jax.ShapeDtypeStruct((NC, N, HD), jnp.float32) if split_out
                 else jax.ShapeDtypeStruct((N, D), jnp.float32))
    return pl.pallas_call(
        _tc_layer_body(split_in, split_out),
        out_shape=out_shape,
        in_specs=[pl.BlockSpec(memory_space=pltpu.SMEM)] + [vmem] * 8,
        out_specs=vmem,
    )(eps.reshape(1, 1), xin, agg, w1, b1.reshape(1, D), w2,
      b2.reshape(1, D), g.reshape(1, D), b.reshape(1, D))


def kernel(x, edge_index, eps1, W11, b11, W12, b12,
           eps2, W21, b21, W22, b22, bn1_g, bn1_b, bn2_g, bn2_b):
    src = edge_index[0].astype(jnp.int32)
    dst = edge_index[1].astype(jnp.int32)
    pad = EPW - E // NS
    src3 = jnp.pad(src.reshape(NS, E // NS), ((0, 0), (0, pad)))
    dst3 = jnp.pad(dst.reshape(NS, E // NS), ((0, 0), (0, pad)),
                   constant_values=N)
    src4 = jnp.stack([src3, src3 + N]).reshape(NC, NS, CPW, CHUNK)
    dst3 = dst3.reshape(NS, CPW, CHUNK)
    zeros = jnp.zeros((CHUNK, HD), jnp.float32)

    x2 = x.reshape(N, NC, HD).transpose(1, 0, 2).reshape(NC * N, HD)

    agg1 = _sc_agg(x2, src4, dst3, zeros)
    h1 = _tc_layer(x, agg1, eps1, W11, b11, W12, b12, bn1_g, bn1_b,
                   split_in=False, split_out=True)
    agg2 = _sc_agg(h1.reshape(NC * N, HD), src4, dst3, zeros)
    return _tc_layer(h1, agg2, eps2, W21, b21, W22, b22, bn2_g, bn2_b,
                     split_in=True, split_out=False)


# speedup vs baseline: 5.9583x; 1.0983x over previous
import jax
import jax.numpy as jnp
from jax import lax
from jax.experimental import pallas as pl
from jax.experimental.pallas import tpu as pltpu
from jax.experimental.pallas import tpu_sc as plsc

N = 10000
D = 128
HD = D // 2
E = 320000
NC, NS = 2, 16
CHUNK = 128
CPW = 160
NP = 3
NB = 6
EPW = CPW * CHUNK
ACC_ROWS = 10112
RPT = ACC_ROWS // NS


def _sc_agg_body(feat_h, src_h, dst_h, zeros_h, out_h,
                 acc, src_v, dst_v, rows_all,
                 sem0, sem1, sem2, sem3, sem4, sem5):
    rows = tuple(rows_all.at[pl.ds(k * CHUNK, CHUNK)] for k in range(NB))
    sems = (sem0, sem1, sem2, sem3, sem4, sem5)
    c = lax.axis_index("c")
    s = lax.axis_index("s")
    pltpu.sync_copy(zeros_h, rows[0])
    for q in range(RPT // CHUNK):
        pltpu.sync_copy(rows[0], acc.at[pl.ds(s * RPT + q * CHUNK, CHUNK)])
    rem = RPT % CHUNK
    if rem:
        pltpu.sync_copy(rows[0].at[pl.ds(0, rem)],
                        acc.at[pl.ds(s * RPT + RPT - rem, rem)])
    pltpu.sync_copy(src_h.at[c, s], src_v)
    pltpu.sync_copy(dst_h.at[s], dst_v)
    plsc.subcore_barrier()

    def visit(j, p, k, reissue):
        b = 2 * p + k
        pltpu.make_async_copy(feat_h.at[src_v.at[j]], rows[b], sems[b]).wait()
        pltpu.sync_copy(rows[b], acc.at[dst_v.at[j]], add=True)
        if reissue:
            q = (p + 2) % NP
            pltpu.async_copy(feat_h.at[src_v.at[j + 4]],
                             rows[2 * q + k], sems[2 * q + k])

    for g in range(2):
        for k in range(2):
            b = 2 * g + k
            pltpu.async_copy(feat_h.at[src_v.at[2 * g + k]], rows[b], sems[b])

    def iteration(i, carry):
        for gg in range(NP):
            for k in range(2):
                visit(2 * (NP * i + gg) + k, gg, k, True)
        return carry

    n_groups = CPW // 2
    lax.fori_loop(0, (n_groups - 2) // NP, iteration, 0)
    for g in range(n_groups - 2, n_groups):
        for k in range(2):
            visit(2 * g + k, g % NP, k, False)

    plsc.subcore_barrier()
    for off, sz in ([(0, NB * CHUNK), (NB * CHUNK, RPT - NB * CHUNK)]
                    if RPT > NB * CHUNK else [(0, RPT)]):
        pltpu.sync_copy(acc.at[pl.ds(s * RPT + off, sz)],
                        rows_all.at[pl.ds(0, sz)])
        pltpu.sync_copy(rows_all.at[pl.ds(0, sz)],
                        out_h.at[c, pl.ds(s * RPT + off, sz)])


def _sc_agg(feat2, src4, dst3, zeros):
    mesh = plsc.VectorSubcoreMesh(
        core_axis_name="c", subcore_axis_name="s")
    fn = pl.kernel(
        _sc_agg_body,
        out_type=jax.ShapeDtypeStruct((NC, ACC_ROWS, HD), jnp.float32),
        mesh=mesh,
        scratch_types=[
            pltpu.VMEM_SHARED((ACC_ROWS, HD), jnp.float32),
            pltpu.VMEM((CPW, CHUNK), jnp.int32),
            pltpu.VMEM((CPW, CHUNK), jnp.int32),
            pltpu.VMEM((NB * CHUNK, HD), jnp.float32),
        ] + [pltpu.SemaphoreType.DMA] * NB,
        compiler_params=pltpu.CompilerParams(use_tc_tiling_on_sc=False),
    )
    return fn(feat2, src4, dst3, zeros)


def _tc_layer_body(split_in, split_out):
    def body(eps_r, x_r, agg_r, w1_r, b1_r, w2_r, b2_r, g_r, b_r, o_r):
        scale = 1.0 + eps_r[0, 0]
        if split_in:
            xv = jnp.concatenate([x_r[0], x_r[1]], axis=1)
        else:
            xv = x_r[...]
        agg = jnp.concatenate([agg_r[0, :N, :], agg_r[1, :N, :]], axis=1)
        h = scale * xv + agg
        h = jnp.dot(h, w1_r[...], preferred_element_type=jnp.float32,
                    precision=lax.Precision.DEFAULT) + b1_r[...]
        h = jnp.maximum(h, 0.0)
        h = jnp.dot(h, w2_r[...], preferred_element_type=jnp.float32,
                    precision=lax.Precision.DEFAULT) + b2_r[...]
        mu = jnp.mean(h, axis=0, keepdims=True)
        var = jnp.mean((h - mu) ** 2, axis=0, keepdims=True)
        o = (h - mu) * lax.rsqrt(var + 1e-5) * g_r[...] + b_r[...]
        o = jnp.maximum(o, 0.0)
        if split_out:
            o_r[0] = o[:, :HD]
            o_r[1] = o[:, HD:]
        else:
            o_r[...] = o
    return body


def _tc_layer(xin, agg, eps, w1, b1, w2, b2, g, b, split_in, split_out):
    vmem = pl.BlockSpec(memory_space=pltpu.VMEM)
    out_shape = (jax.ShapeDtypeStruct((NC, N, HD), jnp.float32) if split_out
                 else jax.ShapeDtypeStruct((N, D), jnp.float32))
    return pl.pallas_call(
        _tc_layer_body(split_in, split_out),
        out_shape=out_shape,
        in_specs=[pl.BlockSpec(memory_space=pltpu.SMEM)] + [vmem] * 8,
        out_specs=vmem,
    )(eps.reshape(1, 1), xin, agg, w1, b1.reshape(1, D), w2,
      b2.reshape(1, D), g.reshape(1, D), b.reshape(1, D))


def kernel(x, edge_index, eps1, W11, b11, W12, b12,
           eps2, W21, b21, W22, b22, bn1_g, bn1_b, bn2_g, bn2_b):
    src = edge_index[0].astype(jnp.int32)
    dst = edge_index[1].astype(jnp.int32)
    pad = EPW - E // NS
    src3 = jnp.pad(src.reshape(NS, E // NS), ((0, 0), (0, pad)))
    dst3 = jnp.pad(dst.reshape(NS, E // NS), ((0, 0), (0, pad)),
                   constant_values=N)
    src4 = jnp.stack([src3, src3 + N]).reshape(NC, NS, CPW, CHUNK)
    dst3 = dst3.reshape(NS, CPW, CHUNK)
    zeros = jnp.zeros((CHUNK, HD), jnp.float32)

    x2 = x.reshape(N, NC, HD).transpose(1, 0, 2).reshape(NC * N, HD)

    agg1 = _sc_agg(x2, src4, dst3, zeros)
    h1 = _tc_layer(x, agg1, eps1, W11, b11, W12, b12, bn1_g, bn1_b,
                   split_in=False, split_out=True)
    agg2 = _sc_agg(h1.reshape(NC * N, HD), src4, dst3, zeros)
    return _tc_layer(h1, agg2, eps2, W21, b21, W22, b22, bn2_g, bn2_b,
                     split_in=True, split_out=False)
